# baseline (speedup 1.0000x reference)
import functools

import jax
import jax.numpy as jnp
from jax import lax
from jax.experimental import pallas as pl
from jax.experimental.pallas import tpu as pltpu
from jax.experimental.pallas import tpu_sc as plsc

_LANES = 16


def _router_body(xf_ref, wr_ref, br_ref, sbits_ref):
    logits = lax.dot_general(
        wr_ref[...], xf_ref[...],
        dimension_numbers=(((0,), (1,)), ((), ())),
        preferred_element_type=jnp.float32) + br_ref[...]
    m = jnp.max(logits, axis=0, keepdims=True)
    ex = jnp.exp(logits - m)
    s = ex / jnp.sum(ex, axis=0, keepdims=True)
    sbits_ref[...] = lax.bitcast_convert_type(s, jnp.int32)


def _make_sc_topk(E, bs, k):
    nvec = bs // _LANES
    mesh = plsc.VectorSubcoreMesh(core_axis_name="c", subcore_axis_name="s")

    def body(s_hbm, w_hbm, s_v, w_v, hist_v):
        cid = lax.axis_index("c")
        sid = lax.axis_index("s")
        wid = sid * 2 + cid

        @pl.when(wid < E)
        def _():
            pltpu.sync_copy(s_hbm.at[wid], s_v)
            zero = jnp.zeros((_LANES,), jnp.int32)
            onev = jnp.full((_LANES,), 1, jnp.int32)
            lane = lax.iota(jnp.int32, _LANES)

            def digit_pass(carry, shift, nbits):
                prefix, kk = carry
                hi = shift + nbits

                def clear(i, _):
                    hist_v[pl.ds(i * _LANES, _LANES)] = zero
                    return 0
                lax.fori_loop(0, 256 // _LANES, clear, 0)

                def accum(i, _):
                    v = s_v[pl.ds(i * _LANES, _LANES)]
                    dig = (v >> shift) & 255
                    inc = lax.shift_right_logical(v, hi) == prefix
                    plsc.addupdate_scatter(hist_v, [dig], onev, mask=inc)
                    return 0
                lax.fori_loop(0, nvec, accum, 0)

                def scan(j, c):
                    total, found, dstar, s_above = c
                    vh = (256 // _LANES) - 1 - j
                    h = hist_v[pl.ds(vh * _LANES, _LANES)]
                    hr = lax.rev(h, (0,))
                    rc = plsc.cumsum(hr)
                    sfx = total + rc
                    m = sfx >= kk
                    anym = jnp.max(plsc.all_reduce_population_count(m)) > 0
                    ffs = jnp.max(plsc.all_reduce_ffs(m))
                    dcand = vh * _LANES + (_LANES - 1) - ffs
                    sa_v = jnp.where(lane == ffs, sfx - hr, zero)
                    sacand = jnp.max(sa_v)
                    upd = jnp.logical_and(jnp.logical_not(found), anym)
                    dstar = jnp.where(upd, dcand, dstar)
                    s_above = jnp.where(upd, sacand, s_above)
                    return (total + jnp.max(rc),
                            jnp.logical_or(found, anym), dstar, s_above)

                _, _, dstar, s_above = lax.fori_loop(
                    0, 256 // _LANES, scan,
                    (jnp.int32(0), jnp.bool_(False), jnp.int32(0),
                     jnp.int32(0)))
                return ((prefix << nbits) | dstar, kk - s_above)

            carry = (jnp.int32(0), jnp.int32(k))
            for shift, nbits in ((23, 8), (15, 8), (7, 8), (0, 7)):
                carry = digit_pass(carry, shift, nbits)
            thr, r = carry

            def emit(i, run):
                v = s_v[pl.ds(i * _LANES, _LANES)]
                gt = v > thr
                eq = v == thr
                eqc = plsc.cumsum(eq.astype(jnp.int32))
                inc = gt | (eq & ((run + eqc) <= r))
                w_v[pl.ds(i * _LANES, _LANES)] = jnp.where(inc, v, zero)
                return run + jnp.max(plsc.all_reduce_population_count(eq))

            lax.fori_loop(0, nvec, emit, jnp.int32(0))
            pltpu.sync_copy(w_v, w_hbm.at[wid])

    return pl.kernel(
        body,
        out_type=jax.ShapeDtypeStruct((E, bs), jnp.int32),
        mesh=mesh,
        compiler_params=pltpu.CompilerParams(needs_layout_passes=False),
        scratch_types=[
            pltpu.VMEM((bs,), jnp.int32),
            pltpu.VMEM((bs,), jnp.int32),
            pltpu.VMEM((256,), jnp.int32),
        ],
    )


def _ffn_body(xb_ref, we_ref, be_ref, y0_ref):
    y0 = jnp.dot(xb_ref[...], we_ref[...],
                 preferred_element_type=jnp.float32) + be_ref[...]
    y0_ref[...] = y0.astype(jnp.bfloat16)


def _scale_body(y0_ref, w8b_ref, y_ref):
    w8 = lax.bitcast_convert_type(w8b_ref[...], jnp.float32)
    ones = jnp.ones((w8.shape[0], 1), jnp.float32)
    w = lax.dot_general(w8, ones,
                        dimension_numbers=(((0,), (0,)), ((), ())),
                        preferred_element_type=jnp.float32)
    y_ref[...] = y0_ref[...].astype(jnp.float32) * w


def kernel(x, W_r, b_r, W_e, b_e):
    b, s, h = x.shape
    bs = b * s
    E = W_r.shape[1]
    k = min(int(bs * 1.0 / E), bs)
    xf = x.reshape(bs, h)

    s_bits = pl.pallas_call(
        _router_body,
        out_shape=jax.ShapeDtypeStruct((E, bs), jnp.int32),
    )(xf, W_r, b_r.reshape(E, 1))

    w8_bits = _make_sc_topk(E, bs, k)(s_bits)

    m_blk = 256
    grid = (bs // m_blk,)
    xb = xf.astype(jnp.bfloat16)
    web = W_e.astype(jnp.bfloat16)
    y0 = pl.pallas_call(
        _ffn_body,
        grid=grid,
        in_specs=[
            pl.BlockSpec((m_blk, h), lambda i: (i, 0)),
            pl.BlockSpec((h, h), lambda i: (0, 0)),
            pl.BlockSpec((1, h), lambda i: (0, 0)),
        ],
        out_specs=pl.BlockSpec((m_blk, h), lambda i: (i, 0)),
        out_shape=jax.ShapeDtypeStruct((bs, h), jnp.bfloat16),
    )(xb, web, b_e.reshape(1, h))

    y = pl.pallas_call(
        _scale_body,
        grid=grid,
        in_specs=[
            pl.BlockSpec((m_blk, h), lambda i: (i, 0)),
            pl.BlockSpec((E, m_blk), lambda i: (0, i)),
        ],
        out_specs=pl.BlockSpec((m_blk, h), lambda i: (i, 0)),
        out_shape=jax.ShapeDtypeStruct((bs, h), jnp.float32),
    )(y0, w8_bits)

    return y.reshape(b, s, h)


# speedup vs baseline: 1.2072x; 1.2072x over previous
import functools

import jax
import jax.numpy as jnp
from jax import lax
from jax.experimental import pallas as pl
from jax.experimental.pallas import tpu as pltpu
from jax.experimental.pallas import tpu_sc as plsc

_LANES = 16


def _router_body(xf_ref, wr_ref, br_ref, sbits_ref):
    logits = lax.dot_general(
        wr_ref[...], xf_ref[...],
        dimension_numbers=(((0,), (1,)), ((), ())),
        preferred_element_type=jnp.float32) + br_ref[...]
    m = jnp.max(logits, axis=0, keepdims=True)
    ex = jnp.exp(logits - m)
    s = ex / jnp.sum(ex, axis=0, keepdims=True)
    sbits_ref[...] = lax.bitcast_convert_type(s, jnp.int32)


def _make_sc_topk(E, bs, k):
    nvec = bs // _LANES
    mesh = plsc.VectorSubcoreMesh(core_axis_name="c", subcore_axis_name="s")

    def body(s_hbm, w_hbm, s_v, w_v, hist_v):
        cid = lax.axis_index("c")
        sid = lax.axis_index("s")
        wid = sid * 2 + cid

        @pl.when(wid < E)
        def _():
            pltpu.sync_copy(s_hbm.at[wid], s_v)
            zero = jnp.zeros((_LANES,), jnp.int32)
            onev = jnp.full((_LANES,), 1, jnp.int32)
            lane = lax.iota(jnp.int32, _LANES)

            def digit_pass(carry, shift, nbits):
                prefix, kk = carry
                hi = shift + nbits

                def clear(i, _):
                    hist_v[pl.ds(i * _LANES, _LANES)] = zero
                    return 0
                lax.fori_loop(0, 256 // _LANES, clear, 0)

                def accum(i, _):
                    v = s_v[pl.ds(i * _LANES, _LANES)]
                    dig = (v >> shift) & 255
                    inc = lax.shift_right_logical(v, hi) == prefix
                    plsc.addupdate_scatter(hist_v, [dig], onev, mask=inc)
                    return 0
                lax.fori_loop(0, nvec, accum, 0)

                def scan(j, c):
                    total, found, dstar, s_above = c
                    vh = (256 // _LANES) - 1 - j
                    h = hist_v[pl.ds(vh * _LANES, _LANES)]
                    hr = lax.rev(h, (0,))
                    rc = plsc.cumsum(hr)
                    sfx = total + rc
                    m = sfx >= kk
                    anym = jnp.max(plsc.all_reduce_population_count(m)) > 0
                    ffs = jnp.max(plsc.all_reduce_ffs(m))
                    dcand = vh * _LANES + (_LANES - 1) - ffs
                    sa_v = jnp.where(lane == ffs, sfx - hr, zero)
                    sacand = jnp.max(sa_v)
                    upd = jnp.logical_and(jnp.logical_not(found), anym)
                    dstar = jnp.where(upd, dcand, dstar)
                    s_above = jnp.where(upd, sacand, s_above)
                    return (total + jnp.max(rc),
                            jnp.logical_or(found, anym), dstar, s_above)

                _, _, dstar, s_above = lax.fori_loop(
                    0, 256 // _LANES, scan,
                    (jnp.int32(0), jnp.bool_(False), jnp.int32(0),
                     jnp.int32(0)))
                return ((prefix << nbits) | dstar, kk - s_above)

            carry = (jnp.int32(0), jnp.int32(k))
            for shift, nbits in ((23, 8), (15, 8), (7, 8), (0, 7)):
                carry = digit_pass(carry, shift, nbits)
            thr, r = carry

            def emit(i, run):
                v = s_v[pl.ds(i * _LANES, _LANES)]
                gt = v > thr
                eq = v == thr
                eqc = plsc.cumsum(eq.astype(jnp.int32))
                inc = gt | (eq & ((run + eqc) <= r))
                w_v[pl.ds(i * _LANES, _LANES)] = jnp.where(inc, v, zero)
                return run + jnp.max(plsc.all_reduce_population_count(eq))

            lax.fori_loop(0, nvec, emit, jnp.int32(0))
            pltpu.sync_copy(w_v, w_hbm.at[wid])

    return pl.kernel(
        body,
        out_type=jax.ShapeDtypeStruct((E, bs), jnp.int32),
        mesh=mesh,
        compiler_params=pltpu.CompilerParams(needs_layout_passes=False),
        scratch_types=[
            pltpu.VMEM((bs,), jnp.int32),
            pltpu.VMEM((bs,), jnp.int32),
            pltpu.VMEM((256,), jnp.int32),
        ],
    )


def _ffn_body(xf_ref, we_ref, be_ref, y0_ref):
    xb = xf_ref[...].astype(jnp.bfloat16)
    web = we_ref[...].astype(jnp.bfloat16)
    y0 = jnp.dot(xb, web,
                 preferred_element_type=jnp.float32) + be_ref[...]
    y0_ref[...] = y0.astype(jnp.bfloat16)


def _scale_body(y0_ref, w8b_ref, y_ref):
    w8 = lax.bitcast_convert_type(w8b_ref[...], jnp.float32)
    ones = jnp.ones((w8.shape[0], 1), jnp.float32)
    w = lax.dot_general(w8, ones,
                        dimension_numbers=(((0,), (0,)), ((), ())),
                        preferred_element_type=jnp.float32)
    y_ref[...] = y0_ref[...].astype(jnp.float32) * w


def kernel(x, W_r, b_r, W_e, b_e):
    b, s, h = x.shape
    bs = b * s
    E = W_r.shape[1]
    k = min(int(bs * 1.0 / E), bs)
    xf = x.reshape(bs, h)

    s_bits = pl.pallas_call(
        _router_body,
        out_shape=jax.ShapeDtypeStruct((E, bs), jnp.int32),
    )(xf, W_r, b_r.reshape(E, 1))

    w8_bits = _make_sc_topk(E, bs, k)(s_bits)

    m_blk = 512
    grid = (bs // m_blk,)
    y0 = pl.pallas_call(
        _ffn_body,
        grid=grid,
        in_specs=[
            pl.BlockSpec((m_blk, h), lambda i: (i, 0)),
            pl.BlockSpec((h, h), lambda i: (0, 0)),
            pl.BlockSpec((1, h), lambda i: (0, 0)),
        ],
        out_specs=pl.BlockSpec((m_blk, h), lambda i: (i, 0)),
        out_shape=jax.ShapeDtypeStruct((bs, h), jnp.bfloat16),
    )(xf, W_e, b_e.reshape(1, h))

    y = pl.pallas_call(
        _scale_body,
        grid=grid,
        in_specs=[
            pl.BlockSpec((m_blk, h), lambda i: (i, 0)),
            pl.BlockSpec((E, m_blk), lambda i: (0, i)),
        ],
        out_specs=pl.BlockSpec((m_blk, h), lambda i: (i, 0)),
        out_shape=jax.ShapeDtypeStruct((bs, h), jnp.float32),
    )(y0, w8_bits)

    return y.reshape(b, s, h)


# speedup vs baseline: 1.2584x; 1.0424x over previous
import functools

import jax
import jax.numpy as jnp
from jax import lax
from jax.experimental import pallas as pl
from jax.experimental.pallas import tpu as pltpu
from jax.experimental.pallas import tpu_sc as plsc

_LANES = 16


def _router_body(xf_ref, wr_ref, br_ref, sbits_ref):
    logits = lax.dot_general(
        wr_ref[...], xf_ref[...],
        dimension_numbers=(((0,), (1,)), ((), ())),
        preferred_element_type=jnp.float32) + br_ref[...]
    m = jnp.max(logits, axis=0, keepdims=True)
    ex = jnp.exp(logits - m)
    s = ex / jnp.sum(ex, axis=0, keepdims=True)
    sbits_ref[...] = lax.bitcast_convert_type(s, jnp.int32)


def _make_sc_topk(E, bs, k):
    NQ = 4
    Q = bs // NQ
    nvq = Q // _LANES
    mesh = plsc.VectorSubcoreMesh(core_axis_name="c", subcore_axis_name="s")

    def body(s_hbm, w_hbm, s_q, w_q, hist_v, hist4_v, shared):
        cid = lax.axis_index("c")
        sid = lax.axis_index("s")
        e_local = sid // NQ
        q = sid % NQ
        expert = cid * (E // 2) + e_local

        pltpu.sync_copy(s_hbm.at[expert, pl.ds(q * Q, Q)], s_q)
        zero = jnp.zeros((_LANES,), jnp.int32)
        onev = jnp.full((_LANES,), 1, jnp.int32)
        lane = lax.iota(jnp.int32, _LANES)

        def digit_pass(carry, shift, nbits):
            prefix, kk = carry
            hi = shift + nbits

            def clear(i, _):
                hist_v[pl.ds(i * _LANES, _LANES)] = zero
                return 0
            lax.fori_loop(0, 256 // _LANES, clear, 0)

            def accum(i, _):
                v = s_q[pl.ds(i * _LANES, _LANES)]
                dig = (v >> shift) & 255
                inc = lax.shift_right_logical(v, hi) == prefix
                plsc.addupdate_scatter(hist_v, [dig], onev, mask=inc)
                return 0
            lax.fori_loop(0, nvq, accum, 0)

            pltpu.sync_copy(hist_v, shared.at[pl.ds(sid * 256, 256)])
            plsc.subcore_barrier()
            pltpu.sync_copy(shared.at[pl.ds(e_local * (NQ * 256), NQ * 256)],
                            hist4_v)
            plsc.subcore_barrier()

            def scan(j, c):
                total, found, dstar, s_above = c
                vh = (256 // _LANES) - 1 - j
                h = (hist4_v[pl.ds(vh * _LANES, _LANES)]
                     + hist4_v[pl.ds(256 + vh * _LANES, _LANES)]
                     + hist4_v[pl.ds(512 + vh * _LANES, _LANES)]
                     + hist4_v[pl.ds(768 + vh * _LANES, _LANES)])
                hr = lax.rev(h, (0,))
                rc = plsc.cumsum(hr)
                sfx = total + rc
                m = sfx >= kk
                anym = jnp.max(plsc.all_reduce_population_count(m)) > 0
                ffs = jnp.max(plsc.all_reduce_ffs(m))
                dcand = vh * _LANES + (_LANES - 1) - ffs
                sa_v = jnp.where(lane == ffs, sfx - hr, zero)
                sacand = jnp.max(sa_v)
                upd = jnp.logical_and(jnp.logical_not(found), anym)
                dstar = jnp.where(upd, dcand, dstar)
                s_above = jnp.where(upd, sacand, s_above)
                return (total + jnp.max(rc),
                        jnp.logical_or(found, anym), dstar, s_above)

            _, _, dstar, s_above = lax.fori_loop(
                0, 256 // _LANES, scan,
                (jnp.int32(0), jnp.bool_(False), jnp.int32(0),
                 jnp.int32(0)))
            return ((prefix << nbits) | dstar, kk - s_above)

        carry = (jnp.int32(0), jnp.int32(k))
        for shift, nbits in ((23, 8), (15, 8), (7, 8), (0, 7)):
            carry = digit_pass(carry, shift, nbits)
        thr, r = carry

        def count_eq(i, c):
            v = s_q[pl.ds(i * _LANES, _LANES)]
            eq = v == thr
            return c + jnp.max(plsc.all_reduce_population_count(eq))
        myeq = lax.fori_loop(0, nvq, count_eq, jnp.int32(0))
        hist_v[pl.ds(0, _LANES)] = jnp.full((_LANES,), 1, jnp.int32) * myeq
        pltpu.sync_copy(hist_v, shared.at[pl.ds(sid * 256, 256)])
        plsc.subcore_barrier()
        pltpu.sync_copy(shared.at[pl.ds(e_local * (NQ * 256), NQ * 256)],
                        hist4_v)
        plsc.subcore_barrier()
        run0 = jnp.int32(0)
        for j in range(NQ):
            cj = jnp.max(hist4_v[pl.ds(j * 256, _LANES)])
            run0 = run0 + jnp.where(q > j, cj, 0)

        def emit(i, run):
            v = s_q[pl.ds(i * _LANES, _LANES)]
            gt = v > thr
            eq = v == thr
            eqc = plsc.cumsum(eq.astype(jnp.int32))
            inc = gt | (eq & ((run + eqc) <= r))
            w_q[pl.ds(i * _LANES, _LANES)] = jnp.where(inc, v, zero)
            return run + jnp.max(plsc.all_reduce_population_count(eq))

        lax.fori_loop(0, nvq, emit, run0)
        pltpu.sync_copy(w_q, w_hbm.at[expert, pl.ds(q * Q, Q)])

    return pl.kernel(
        body,
        out_type=jax.ShapeDtypeStruct((E, bs), jnp.int32),
        mesh=mesh,
        compiler_params=pltpu.CompilerParams(needs_layout_passes=False),
        scratch_types=[
            pltpu.VMEM((Q,), jnp.int32),
            pltpu.VMEM((Q,), jnp.int32),
            pltpu.VMEM((256,), jnp.int32),
            pltpu.VMEM((4 * 256,), jnp.int32),
            pltpu.VMEM_SHARED((16 * 256,), jnp.int32),
        ],
    )


def _ffn_body(xf_ref, we_ref, be_ref, y0_ref):
    xb = xf_ref[...].astype(jnp.bfloat16)
    web = we_ref[...].astype(jnp.bfloat16)
    y0 = jnp.dot(xb, web,
                 preferred_element_type=jnp.float32) + be_ref[...]
    y0_ref[...] = y0.astype(jnp.bfloat16)


def _scale_body(y0_ref, w8b_ref, y_ref):
    w8 = lax.bitcast_convert_type(w8b_ref[...], jnp.float32)
    ones = jnp.ones((w8.shape[0], 1), jnp.float32)
    w = lax.dot_general(w8, ones,
                        dimension_numbers=(((0,), (0,)), ((), ())),
                        preferred_element_type=jnp.float32)
    y_ref[...] = y0_ref[...].astype(jnp.float32) * w


def kernel(x, W_r, b_r, W_e, b_e):
    b, s, h = x.shape
    bs = b * s
    E = W_r.shape[1]
    k = min(int(bs * 1.0 / E), bs)
    xf = x.reshape(bs, h)

    s_bits = pl.pallas_call(
        _router_body,
        out_shape=jax.ShapeDtypeStruct((E, bs), jnp.int32),
    )(xf, W_r, b_r.reshape(E, 1))

    w8_bits = _make_sc_topk(E, bs, k)(s_bits)

    m_blk = 512
    grid = (bs // m_blk,)
    y0 = pl.pallas_call(
        _ffn_body,
        grid=grid,
        in_specs=[
            pl.BlockSpec((m_blk, h), lambda i: (i, 0)),
            pl.BlockSpec((h, h), lambda i: (0, 0)),
            pl.BlockSpec((1, h), lambda i: (0, 0)),
        ],
        out_specs=pl.BlockSpec((m_blk, h), lambda i: (i, 0)),
        out_shape=jax.ShapeDtypeStruct((bs, h), jnp.bfloat16),
    )(xf, W_e, b_e.reshape(1, h))

    y = pl.pallas_call(
        _scale_body,
        grid=grid,
        in_specs=[
            pl.BlockSpec((m_blk, h), lambda i: (i, 0)),
            pl.BlockSpec((E, m_blk), lambda i: (0, i)),
        ],
        out_specs=pl.BlockSpec((m_blk, h), lambda i: (i, 0)),
        out_shape=jax.ShapeDtypeStruct((bs, h), jnp.float32),
    )(y0, w8_bits)

    return y.reshape(b, s, h)


# speedup vs baseline: 1.3719x; 1.0901x over previous
import functools

import jax
import jax.numpy as jnp
from jax import lax
from jax.experimental import pallas as pl
from jax.experimental.pallas import tpu as pltpu
from jax.experimental.pallas import tpu_sc as plsc

_LANES = 16


def _router_body(xf_ref, wr_ref, br_ref, sbits_ref):
    logits = lax.dot_general(
        wr_ref[...], xf_ref[...],
        dimension_numbers=(((0,), (1,)), ((), ())),
        preferred_element_type=jnp.float32) + jnp.transpose(br_ref[...])
    m = jnp.max(logits, axis=0, keepdims=True)
    ex = jnp.exp(logits - m)
    s = ex / jnp.sum(ex, axis=0, keepdims=True)
    sbits_ref[...] = lax.bitcast_convert_type(s, jnp.int32)


def _make_sc_topk(E, bs, k):
    NQ = 4
    Q = bs // NQ
    nvq = Q // _LANES
    mesh = plsc.VectorSubcoreMesh(core_axis_name="c", subcore_axis_name="s")

    def body(s_hbm, w_hbm, s_q, w_q, hist_v, hist4_v, shared):
        cid = lax.axis_index("c")
        sid = lax.axis_index("s")
        e_local = sid // NQ
        q = sid % NQ
        expert = cid * (E // 2) + e_local

        pltpu.sync_copy(s_hbm.at[expert, pl.ds(q * Q, Q)], s_q)
        zero = jnp.zeros((_LANES,), jnp.int32)
        onev = jnp.full((_LANES,), 1, jnp.int32)
        lane = lax.iota(jnp.int32, _LANES)

        def digit_pass(carry, shift, nbits):
            prefix, kk = carry
            hi = shift + nbits

            def clear(i, _):
                hist_v[pl.ds(i * _LANES, _LANES)] = zero
                return 0
            lax.fori_loop(0, 256 // _LANES, clear, 0)

            def accum(i, _):
                v = s_q[pl.ds(i * _LANES, _LANES)]
                dig = (v >> shift) & 255
                inc = lax.shift_right_logical(v, hi) == prefix
                plsc.addupdate_scatter(hist_v, [dig], onev, mask=inc)
                return 0
            lax.fori_loop(0, nvq, accum, 0)

            pltpu.sync_copy(hist_v, shared.at[pl.ds(sid * 256, 256)])
            plsc.subcore_barrier()
            pltpu.sync_copy(shared.at[pl.ds(e_local * (NQ * 256), NQ * 256)],
                            hist4_v)
            plsc.subcore_barrier()

            def scan(j, c):
                total, found, dstar, s_above = c
                vh = (256 // _LANES) - 1 - j
                h = (hist4_v[pl.ds(vh * _LANES, _LANES)]
                     + hist4_v[pl.ds(256 + vh * _LANES, _LANES)]
                     + hist4_v[pl.ds(512 + vh * _LANES, _LANES)]
                     + hist4_v[pl.ds(768 + vh * _LANES, _LANES)])
                hr = lax.rev(h, (0,))
                rc = plsc.cumsum(hr)
                sfx = total + rc
                m = sfx >= kk
                anym = jnp.max(plsc.all_reduce_population_count(m)) > 0
                ffs = jnp.max(plsc.all_reduce_ffs(m))
                dcand = vh * _LANES + (_LANES - 1) - ffs
                sa_v = jnp.where(lane == ffs, sfx - hr, zero)
                sacand = jnp.max(sa_v)
                upd = jnp.logical_and(jnp.logical_not(found), anym)
                dstar = jnp.where(upd, dcand, dstar)
                s_above = jnp.where(upd, sacand, s_above)
                return (total + jnp.max(rc),
                        jnp.logical_or(found, anym), dstar, s_above)

            _, _, dstar, s_above = lax.fori_loop(
                0, 256 // _LANES, scan,
                (jnp.int32(0), jnp.bool_(False), jnp.int32(0),
                 jnp.int32(0)))
            return ((prefix << nbits) | dstar, kk - s_above)

        carry = (jnp.int32(0), jnp.int32(k))
        for shift, nbits in ((23, 8), (15, 8), (7, 8), (0, 7)):
            carry = digit_pass(carry, shift, nbits)
        thr, r = carry

        def count_eq(i, c):
            v = s_q[pl.ds(i * _LANES, _LANES)]
            eq = v == thr
            return c + jnp.max(plsc.all_reduce_population_count(eq))
        myeq = lax.fori_loop(0, nvq, count_eq, jnp.int32(0))
        hist_v[pl.ds(0, _LANES)] = jnp.full((_LANES,), 1, jnp.int32) * myeq
        pltpu.sync_copy(hist_v, shared.at[pl.ds(sid * 256, 256)])
        plsc.subcore_barrier()
        pltpu.sync_copy(shared.at[pl.ds(e_local * (NQ * 256), NQ * 256)],
                        hist4_v)
        plsc.subcore_barrier()
        run0 = jnp.int32(0)
        for j in range(NQ):
            cj = jnp.max(hist4_v[pl.ds(j * 256, _LANES)])
            run0 = run0 + jnp.where(q > j, cj, 0)

        def emit(i, run):
            v = s_q[pl.ds(i * _LANES, _LANES)]
            gt = v > thr
            eq = v == thr
            eqc = plsc.cumsum(eq.astype(jnp.int32))
            inc = gt | (eq & ((run + eqc) <= r))
            w_q[pl.ds(i * _LANES, _LANES)] = jnp.where(inc, v, zero)
            return run + jnp.max(plsc.all_reduce_population_count(eq))

        lax.fori_loop(0, nvq, emit, run0)
        pltpu.sync_copy(w_q, w_hbm.at[expert, pl.ds(q * Q, Q)])

    return pl.kernel(
        body,
        out_type=jax.ShapeDtypeStruct((E, bs), jnp.int32),
        mesh=mesh,
        compiler_params=pltpu.CompilerParams(needs_layout_passes=False),
        scratch_types=[
            pltpu.VMEM((Q,), jnp.int32),
            pltpu.VMEM((Q,), jnp.int32),
            pltpu.VMEM((256,), jnp.int32),
            pltpu.VMEM((4 * 256,), jnp.int32),
            pltpu.VMEM_SHARED((16 * 256,), jnp.int32),
        ],
    )


def _ffn_body(xf_ref, we_ref, be_ref, y0_ref):
    xb = xf_ref[...].astype(jnp.bfloat16)
    web = we_ref[...].astype(jnp.bfloat16)
    y0 = jnp.dot(xb, web,
                 preferred_element_type=jnp.float32) + be_ref[...]
    y0_ref[...] = y0.astype(jnp.bfloat16)


def _scale_body(y0_ref, w8b_ref, y_ref):
    w8 = lax.bitcast_convert_type(w8b_ref[...], jnp.float32)
    ones = jnp.ones((w8.shape[0], 1), jnp.float32)
    w = lax.dot_general(w8, ones,
                        dimension_numbers=(((0,), (0,)), ((), ())),
                        preferred_element_type=jnp.float32)
    y_ref[...] = y0_ref[...].astype(jnp.float32) * w


def kernel(x, W_r, b_r, W_e, b_e):
    b, s, h = x.shape
    bs = b * s
    E = W_r.shape[1]
    k = min(int(bs * 1.0 / E), bs)
    xf = x.reshape(bs, h)

    s_bits = pl.pallas_call(
        _router_body,
        out_shape=jax.ShapeDtypeStruct((E, bs), jnp.int32),
    )(xf, W_r, b_r.reshape(1, E))

    w8_bits = _make_sc_topk(E, bs, k)(s_bits)

    m_blk = 1024
    grid = (bs // m_blk,)
    y0 = pl.pallas_call(
        _ffn_body,
        grid=grid,
        in_specs=[
            pl.BlockSpec((m_blk, h), lambda i: (i, 0)),
            pl.BlockSpec((h, h), lambda i: (0, 0)),
            pl.BlockSpec((1, h), lambda i: (0, 0)),
        ],
        out_specs=pl.BlockSpec((m_blk, h), lambda i: (i, 0)),
        out_shape=jax.ShapeDtypeStruct((bs, h), jnp.bfloat16),
    )(xf, W_e, b_e.reshape(1, h))

    y = pl.pallas_call(
        _scale_body,
        grid=grid,
        in_specs=[
            pl.BlockSpec((m_blk, h), lambda i: (i, 0)),
            pl.BlockSpec((E, m_blk), lambda i: (0, i)),
        ],
        out_specs=pl.BlockSpec((m_blk, h), lambda i: (i, 0)),
        out_shape=jax.ShapeDtypeStruct((bs, h), jnp.float32),
    )(y0, w8_bits)

    return y.reshape(b, s, h)


# speedup vs baseline: 1.4408x; 1.0502x over previous
import functools

import jax
import jax.numpy as jnp
from jax import lax
from jax.experimental import pallas as pl
from jax.experimental.pallas import tpu as pltpu
from jax.experimental.pallas import tpu_sc as plsc

_LANES = 16


def _router_body(xf_ref, wrt_ref, br_ref, sbits_ref):
    logits = lax.dot_general(
        wrt_ref[...], xf_ref[...],
        dimension_numbers=(((1,), (1,)), ((), ())),
        preferred_element_type=jnp.float32) + jnp.transpose(br_ref[...])
    m = jnp.max(logits, axis=0, keepdims=True)
    ex = jnp.exp(logits - m)
    s = ex / jnp.sum(ex, axis=0, keepdims=True)
    sbits_ref[...] = lax.bitcast_convert_type(s, jnp.int32)


def _make_sc_topk(E, bs, k):
    NQ = 4
    Q = bs // NQ
    nvq = Q // _LANES
    mesh = plsc.VectorSubcoreMesh(core_axis_name="c", subcore_axis_name="s")

    def body(s_hbm, w_hbm, s_q, w_q, hist_v, hist4_v, shared):
        cid = lax.axis_index("c")
        sid = lax.axis_index("s")
        e_local = sid // NQ
        q = sid % NQ
        expert = cid * (E // 2) + e_local

        pltpu.sync_copy(s_hbm.at[expert, pl.ds(q * Q, Q)], s_q)
        zero = jnp.zeros((_LANES,), jnp.int32)
        onev = jnp.full((_LANES,), 1, jnp.int32)
        lane = lax.iota(jnp.int32, _LANES)

        def digit_pass(carry, shift, nbits):
            prefix, kk = carry
            hi = shift + nbits

            def clear(i, _):
                hist_v[pl.ds(i * _LANES, _LANES)] = zero
                return 0
            lax.fori_loop(0, 256 // _LANES, clear, 0)

            def accum(i, _):
                v = s_q[pl.ds(i * _LANES, _LANES)]
                dig = (v >> shift) & 255
                inc = lax.shift_right_logical(v, hi) == prefix
                plsc.addupdate_scatter(hist_v, [dig], onev, mask=inc)
                return 0
            lax.fori_loop(0, nvq, accum, 0)

            pltpu.sync_copy(hist_v, shared.at[pl.ds(sid * 256, 256)])
            plsc.subcore_barrier()
            pltpu.sync_copy(shared.at[pl.ds(e_local * (NQ * 256), NQ * 256)],
                            hist4_v)
            plsc.subcore_barrier()

            def scan(j, c):
                total, found, dstar, s_above = c
                vh = (256 // _LANES) - 1 - j
                h = (hist4_v[pl.ds(vh * _LANES, _LANES)]
                     + hist4_v[pl.ds(256 + vh * _LANES, _LANES)]
                     + hist4_v[pl.ds(512 + vh * _LANES, _LANES)]
                     + hist4_v[pl.ds(768 + vh * _LANES, _LANES)])
                hr = lax.rev(h, (0,))
                rc = plsc.cumsum(hr)
                sfx = total + rc
                m = sfx >= kk
                anym = jnp.max(plsc.all_reduce_population_count(m)) > 0
                ffs = jnp.max(plsc.all_reduce_ffs(m))
                dcand = vh * _LANES + (_LANES - 1) - ffs
                sa_v = jnp.where(lane == ffs, sfx - hr, zero)
                sacand = jnp.max(sa_v)
                upd = jnp.logical_and(jnp.logical_not(found), anym)
                dstar = jnp.where(upd, dcand, dstar)
                s_above = jnp.where(upd, sacand, s_above)
                return (total + jnp.max(rc),
                        jnp.logical_or(found, anym), dstar, s_above)

            _, _, dstar, s_above = lax.fori_loop(
                0, 256 // _LANES, scan,
                (jnp.int32(0), jnp.bool_(False), jnp.int32(0),
                 jnp.int32(0)))
            return ((prefix << nbits) | dstar, kk - s_above)

        carry = (jnp.int32(0), jnp.int32(k))
        for shift, nbits in ((23, 8), (15, 8), (7, 8), (0, 7)):
            carry = digit_pass(carry, shift, nbits)
        thr, r = carry

        def count_eq(i, c):
            v = s_q[pl.ds(i * _LANES, _LANES)]
            eq = v == thr
            return c + jnp.max(plsc.all_reduce_population_count(eq))
        myeq = lax.fori_loop(0, nvq, count_eq, jnp.int32(0))
        hist_v[pl.ds(0, _LANES)] = jnp.full((_LANES,), 1, jnp.int32) * myeq
        pltpu.sync_copy(hist_v, shared.at[pl.ds(sid * 256, 256)])
        plsc.subcore_barrier()
        pltpu.sync_copy(shared.at[pl.ds(e_local * (NQ * 256), NQ * 256)],
                        hist4_v)
        plsc.subcore_barrier()
        run0 = jnp.int32(0)
        for j in range(NQ):
            cj = jnp.max(hist4_v[pl.ds(j * 256, _LANES)])
            run0 = run0 + jnp.where(q > j, cj, 0)

        def emit(i, run):
            v = s_q[pl.ds(i * _LANES, _LANES)]
            gt = v > thr
            eq = v == thr
            eqc = plsc.cumsum(eq.astype(jnp.int32))
            inc = gt | (eq & ((run + eqc) <= r))
            w_q[pl.ds(i * _LANES, _LANES)] = jnp.where(inc, v, zero)
            return run + jnp.max(plsc.all_reduce_population_count(eq))

        lax.fori_loop(0, nvq, emit, run0)
        pltpu.sync_copy(w_q, w_hbm.at[expert, pl.ds(q * Q, Q)])

    return pl.kernel(
        body,
        out_type=jax.ShapeDtypeStruct((E, bs), jnp.int32),
        mesh=mesh,
        compiler_params=pltpu.CompilerParams(needs_layout_passes=False),
        scratch_types=[
            pltpu.VMEM((Q,), jnp.int32),
            pltpu.VMEM((Q,), jnp.int32),
            pltpu.VMEM((256,), jnp.int32),
            pltpu.VMEM((4 * 256,), jnp.int32),
            pltpu.VMEM_SHARED((16 * 256,), jnp.int32),
        ],
    )


def _ffn_body(xf_ref, we_ref, be_ref, y0_ref):
    xb = xf_ref[...].astype(jnp.bfloat16)
    web = we_ref[...].astype(jnp.bfloat16)
    y0 = jnp.dot(xb, web,
                 preferred_element_type=jnp.float32) + be_ref[...]
    y0_ref[...] = y0.astype(jnp.bfloat16)


def _scale_body(y0_ref, w8b_ref, y_ref):
    w8 = lax.bitcast_convert_type(w8b_ref[...], jnp.float32)
    ones = jnp.ones((w8.shape[0], 1), jnp.float32)
    w = lax.dot_general(w8, ones,
                        dimension_numbers=(((0,), (0,)), ((), ())),
                        preferred_element_type=jnp.float32)
    y_ref[...] = y0_ref[...].astype(jnp.float32) * w


def kernel(x, W_r, b_r, W_e, b_e):
    b, s, h = x.shape
    bs = b * s
    E = W_r.shape[1]
    k = min(int(bs * 1.0 / E), bs)
    xf = x.reshape(bs, h)

    s_bits = pl.pallas_call(
        _router_body,
        out_shape=jax.ShapeDtypeStruct((E, bs), jnp.int32),
    )(xf, W_r.T, b_r.reshape(1, E))

    w8_bits = _make_sc_topk(E, bs, k)(s_bits)

    m_blk = 1024
    grid = (bs // m_blk,)
    y0 = pl.pallas_call(
        _ffn_body,
        grid=grid,
        in_specs=[
            pl.BlockSpec((m_blk, h), lambda i: (i, 0)),
            pl.BlockSpec((h, h), lambda i: (0, 0)),
            pl.BlockSpec((1, h), lambda i: (0, 0)),
        ],
        out_specs=pl.BlockSpec((m_blk, h), lambda i: (i, 0)),
        out_shape=jax.ShapeDtypeStruct((bs, h), jnp.bfloat16),
    )(xf, W_e, b_e.reshape(1, h))

    y = pl.pallas_call(
        _scale_body,
        grid=grid,
        in_specs=[
            pl.BlockSpec((m_blk, h), lambda i: (i, 0)),
            pl.BlockSpec((E, m_blk), lambda i: (0, i)),
        ],
        out_specs=pl.BlockSpec((m_blk, h), lambda i: (i, 0)),
        out_shape=jax.ShapeDtypeStruct((bs, h), jnp.float32),
    )(y0, w8_bits)

    return y.reshape(b, s, h)
